# baseline (speedup 1.0000x reference)
import jax
import jax.numpy as jnp
from jax.experimental import pallas as pl
from jax.experimental.pallas import tpu as pltpu

_ROWS = 1000000
_DIM = 64
_NCHUNK = 8
_CHUNK = _ROWS // _NCHUNK


def _body(idx_ref, emb_ref, new_ref, out_ref, sem, row_sem):
    copies = []
    for c in range(_NCHUNK):
        cp = pltpu.make_async_copy(
            emb_ref.at[pl.ds(c * _CHUNK, _CHUNK), :],
            out_ref.at[pl.ds(c * _CHUNK, _CHUNK), :],
            sem,
        )
        cp.start()
        copies.append(cp)
    for cp in copies:
        cp.wait()
    idx = idx_ref[0]
    rcp = pltpu.make_async_copy(new_ref, out_ref.at[pl.ds(idx, 1), :], row_sem)
    rcp.start()
    rcp.wait()


def kernel(embeddings, emb_index, new_emb):
    idx = jnp.asarray(emb_index, jnp.int32).reshape(1)
    new_row = new_emb.reshape(1, _DIM)
    return pl.pallas_call(
        _body,
        in_specs=[
            pl.BlockSpec(memory_space=pltpu.SMEM),
            pl.BlockSpec(memory_space=pl.ANY),
            pl.BlockSpec(memory_space=pl.ANY),
        ],
        out_specs=pl.BlockSpec(memory_space=pl.ANY),
        out_shape=jax.ShapeDtypeStruct((_ROWS, _DIM), embeddings.dtype),
        scratch_shapes=[pltpu.SemaphoreType.DMA, pltpu.SemaphoreType.DMA],
    )(idx, embeddings, new_row)
